# baseline (speedup 1.0000x reference)
import jax
import jax.numpy as jnp
from jax import lax
from jax.experimental import pallas as pl
from jax.experimental.pallas import tpu as pltpu
from jax.experimental.pallas import tpu_sc as plsc

_W = 256
_WB = 1024
_NW = 32
_TBLK = 2048
_YBLK = 2048

_MESH = plsc.VectorSubcoreMesh(core_axis_name="core",
                               subcore_axis_name="subcore")
_SC_PARAMS = pltpu.CompilerParams(use_tc_tiling_on_sc=False)


def _transpose_table(table_t):
    dim, v = table_t.shape
    grid = (v + _TBLK - 1) // _TBLK

    def body(in_ref, out_ref):
        out_ref[:, :dim] = in_ref[...].T

    return pl.pallas_call(
        body,
        grid=(grid,),
        in_specs=[pl.BlockSpec((dim, _TBLK), lambda g: (0, g))],
        out_specs=pl.BlockSpec((_TBLK, 128), lambda g: (g, 0)),
        out_shape=jax.ShapeDtypeStruct((v, 128), table_t.dtype),
        compiler_params=pltpu.CompilerParams(
            dimension_semantics=("parallel",)),
    )(table_t)


def _transpose_y(y3, dim, j0, nj):
    n = y3.shape[1]

    def body(in_ref, out_ref):
        out_ref[0] = in_ref[0][:, :dim].T

    return pl.pallas_call(
        body,
        grid=(nj, n // _YBLK),
        in_specs=[pl.BlockSpec((1, _YBLK, 128),
                               lambda j, i: (j + j0, i, 0))],
        out_specs=pl.BlockSpec((1, dim, _YBLK), lambda j, i: (j, 0, i)),
        out_shape=jax.ShapeDtypeStruct((nj, dim, n), y3.dtype),
        compiler_params=pltpu.CompilerParams(
            dimension_semantics=("parallel", "parallel")),
    )(y3)


def _gather_bc(x_flat, b, c, num):
    spw = num // (_WB * _NW)

    @pl.kernel(
        out_type=(
            jax.ShapeDtypeStruct((num,), b.dtype),
            jax.ShapeDtypeStruct((num,), c.dtype),
        ),
        mesh=_MESH,
        scratch_types=[
            pltpu.VMEM((_WB,), jnp.int32), pltpu.VMEM((_WB,), jnp.int32),
            pltpu.VMEM((_WB,), jnp.float32), pltpu.VMEM((_WB,), jnp.float32),
            pltpu.VMEM((_WB,), jnp.float32), pltpu.VMEM((_WB,), jnp.float32),
            pltpu.SemaphoreType.DMA, pltpu.SemaphoreType.DMA,
            pltpu.SemaphoreType.DMA, pltpu.SemaphoreType.DMA,
        ],
        compiler_params=_SC_PARAMS,
    )
    def bc_kernel(x_hbm, b_hbm, c_hbm, bo_hbm, co_hbm,
                  idx_a, idx_b, b_a, b_b, c_a, c_b,
                  sem_g, sem_oa, sem_ob, sem_i):
        wid = lax.axis_index("subcore") * 2 + lax.axis_index("core")
        base = wid * spw * _WB
        idx_bufs = (idx_a, idx_b)
        b_bufs = (b_a, b_b)
        c_bufs = (c_a, c_b)
        out_sems = (sem_oa, sem_ob)
        pending = [None, None]

        pltpu.async_copy(x_hbm.at[pl.ds(base, _WB)], idx_a, sem_i).wait()
        for s in range(spw):
            p = s % 2
            ib, bb, cb = idx_bufs[p], b_bufs[p], c_bufs[p]
            cp_i = None
            if s + 1 < spw:
                cp_i = pltpu.async_copy(
                    x_hbm.at[pl.ds(base + (s + 1) * _WB, _WB)],
                    idx_bufs[1 - p], sem_i)
            if pending[p] is not None:
                for h in pending[p]:
                    h.wait()
            g_b = pltpu.async_copy(b_hbm.at[ib], bb, sem_g)
            g_c = pltpu.async_copy(c_hbm.at[ib], cb, sem_g)
            g_b.wait()
            g_c.wait()
            off = base + s * _WB
            pending[p] = (
                pltpu.async_copy(bb, bo_hbm.at[pl.ds(off, _WB)], out_sems[p]),
                pltpu.async_copy(cb, co_hbm.at[pl.ds(off, _WB)], out_sems[p]),
            )
            if cp_i is not None:
                cp_i.wait()
        for pend in pending:
            if pend is not None:
                for h in pend:
                    h.wait()

    return bc_kernel(x_flat, b, c)


def _gather_rows(x_half, table_rm, half):
    spw = half // (_W * _NW)

    @pl.kernel(
        out_type=jax.ShapeDtypeStruct((half, 128), table_rm.dtype),
        mesh=_MESH,
        scratch_types=[
            pltpu.VMEM((_W,), jnp.int32), pltpu.VMEM((_W,), jnp.int32),
            pltpu.VMEM((_W, 128), jnp.float32),
            pltpu.VMEM((_W, 128), jnp.float32),
            pltpu.SemaphoreType.DMA, pltpu.SemaphoreType.DMA,
            pltpu.SemaphoreType.DMA, pltpu.SemaphoreType.DMA,
        ],
        compiler_params=_SC_PARAMS,
    )
    def row_kernel(x_hbm, table_hbm, y_hbm,
                   idx_a, idx_b, y_a, y_b, sem_g, sem_oa, sem_ob, sem_i):
        wid = lax.axis_index("subcore") * 2 + lax.axis_index("core")
        base = wid * spw * _W
        idx_bufs = (idx_a, idx_b)
        y_bufs = (y_a, y_b)
        out_sems = (sem_oa, sem_ob)
        pending = [None, None]

        pltpu.async_copy(x_hbm.at[pl.ds(base, _W)], idx_a, sem_i).wait()
        for s in range(spw):
            p = s % 2
            ib, yb = idx_bufs[p], y_bufs[p]
            cp_i = None
            if s + 1 < spw:
                cp_i = pltpu.async_copy(
                    x_hbm.at[pl.ds(base + (s + 1) * _W, _W)],
                    idx_bufs[1 - p], sem_i)
            if pending[p] is not None:
                pending[p].wait()
            pltpu.async_copy(table_hbm.at[ib], yb, sem_g).wait()
            off = base + s * _W
            pending[p] = pltpu.async_copy(
                yb, y_hbm.at[pl.ds(off, _W), :], out_sems[p])
            if cp_i is not None:
                cp_i.wait()
        for pend in pending:
            if pend is not None:
                pend.wait()

    return row_kernel(x_half, table_rm)


def kernel(x, table, b, c):
    n, k = x.shape
    num = n * k
    v, dim = table.shape
    half = num // 2
    kh = k // 2

    x_flat = x.T.reshape(num)

    bsc, csc = _gather_bc(x_flat, b, c, num)

    table_rm = _transpose_table(table.T)

    del half, kh
    ysc = _gather_rows(x_flat, table_rm, num)
    y_p = _transpose_y(ysc.reshape(k, n, 128), dim, 0, k)

    y = jnp.transpose(y_p, (2, 0, 1))
    b_out = bsc.reshape(k, n).T
    c_out = csc.reshape(k, n).T
    return (y, b_out, c_out)


# speedup vs baseline: 1.2822x; 1.2822x over previous
import jax
import jax.numpy as jnp
from jax import lax
from jax.experimental import pallas as pl
from jax.experimental.pallas import tpu as pltpu
from jax.experimental.pallas import tpu_sc as plsc

_W = 256
_WB = 1024
_NW = 32
_TBLK = 4096
_YBLK = 4096

_MESH = plsc.VectorSubcoreMesh(core_axis_name="core",
                               subcore_axis_name="subcore")
_SC_PARAMS = pltpu.CompilerParams(use_tc_tiling_on_sc=False)


def _transpose_table(table_t):
    dim, v = table_t.shape
    grid = (v + _TBLK - 1) // _TBLK

    def body(in_ref, out_ref):
        out_ref[:, :dim] = in_ref[...].T

    return pl.pallas_call(
        body,
        grid=(grid,),
        in_specs=[pl.BlockSpec((dim, _TBLK), lambda g: (0, g))],
        out_specs=pl.BlockSpec((_TBLK, 128), lambda g: (g, 0)),
        out_shape=jax.ShapeDtypeStruct((v, 128), table_t.dtype),
        compiler_params=pltpu.CompilerParams(
            dimension_semantics=("parallel",)),
    )(table_t)


def _transpose_y(y3, dim, j0, nj):
    n = y3.shape[1]

    def body(in_ref, out_ref):
        out_ref[0] = in_ref[0][:, :dim].T

    return pl.pallas_call(
        body,
        grid=(nj, n // _YBLK),
        in_specs=[pl.BlockSpec((1, _YBLK, 128),
                               lambda j, i: (j + j0, i, 0))],
        out_specs=pl.BlockSpec((1, dim, _YBLK), lambda j, i: (j, 0, i)),
        out_shape=jax.ShapeDtypeStruct((nj, dim, n), y3.dtype),
        compiler_params=pltpu.CompilerParams(
            dimension_semantics=("parallel", "parallel")),
    )(y3)


def _gather_bc(x_flat, b, c, num):
    spw = num // (_WB * _NW)

    @pl.kernel(
        out_type=(
            jax.ShapeDtypeStruct((num,), b.dtype),
            jax.ShapeDtypeStruct((num,), c.dtype),
        ),
        mesh=_MESH,
        scratch_types=[
            pltpu.VMEM((_WB,), jnp.int32), pltpu.VMEM((_WB,), jnp.int32),
            pltpu.VMEM((_WB,), jnp.float32), pltpu.VMEM((_WB,), jnp.float32),
            pltpu.VMEM((_WB,), jnp.float32), pltpu.VMEM((_WB,), jnp.float32),
            pltpu.SemaphoreType.DMA, pltpu.SemaphoreType.DMA,
            pltpu.SemaphoreType.DMA, pltpu.SemaphoreType.DMA,
        ],
        compiler_params=_SC_PARAMS,
    )
    def bc_kernel(x_hbm, b_hbm, c_hbm, bo_hbm, co_hbm,
                  idx_a, idx_b, b_a, b_b, c_a, c_b,
                  sem_g, sem_oa, sem_ob, sem_i):
        wid = lax.axis_index("subcore") * 2 + lax.axis_index("core")
        base = wid * spw * _WB
        idx_bufs = (idx_a, idx_b)
        b_bufs = (b_a, b_b)
        c_bufs = (c_a, c_b)
        out_sems = (sem_oa, sem_ob)
        pending = [None, None]

        pltpu.async_copy(x_hbm.at[pl.ds(base, _WB)], idx_a, sem_i).wait()
        for s in range(spw):
            p = s % 2
            ib, bb, cb = idx_bufs[p], b_bufs[p], c_bufs[p]
            cp_i = None
            if s + 1 < spw:
                cp_i = pltpu.async_copy(
                    x_hbm.at[pl.ds(base + (s + 1) * _WB, _WB)],
                    idx_bufs[1 - p], sem_i)
            if pending[p] is not None:
                for h in pending[p]:
                    h.wait()
            g_b = pltpu.async_copy(b_hbm.at[ib], bb, sem_g)
            g_c = pltpu.async_copy(c_hbm.at[ib], cb, sem_g)
            g_b.wait()
            g_c.wait()
            off = base + s * _WB
            pending[p] = (
                pltpu.async_copy(bb, bo_hbm.at[pl.ds(off, _WB)], out_sems[p]),
                pltpu.async_copy(cb, co_hbm.at[pl.ds(off, _WB)], out_sems[p]),
            )
            if cp_i is not None:
                cp_i.wait()
        for pend in pending:
            if pend is not None:
                for h in pend:
                    h.wait()

    return bc_kernel(x_flat, b, c)


def _gather_rows(x_half, table_rm, half):
    spw = half // (_W * _NW)

    @pl.kernel(
        out_type=jax.ShapeDtypeStruct((half, 128), table_rm.dtype),
        mesh=_MESH,
        scratch_types=[
            pltpu.VMEM((_W,), jnp.int32), pltpu.VMEM((_W,), jnp.int32),
            pltpu.VMEM((_W, 128), jnp.float32),
            pltpu.VMEM((_W, 128), jnp.float32),
            pltpu.SemaphoreType.DMA, pltpu.SemaphoreType.DMA,
            pltpu.SemaphoreType.DMA, pltpu.SemaphoreType.DMA,
        ],
        compiler_params=_SC_PARAMS,
    )
    def row_kernel(x_hbm, table_hbm, y_hbm,
                   idx_a, idx_b, y_a, y_b, sem_g, sem_oa, sem_ob, sem_i):
        wid = lax.axis_index("subcore") * 2 + lax.axis_index("core")
        base = wid * spw * _W
        idx_bufs = (idx_a, idx_b)
        y_bufs = (y_a, y_b)
        out_sems = (sem_oa, sem_ob)
        pending = [None, None]

        pltpu.async_copy(x_hbm.at[pl.ds(base, _W)], idx_a, sem_i).wait()
        for s in range(spw):
            p = s % 2
            ib, yb = idx_bufs[p], y_bufs[p]
            cp_i = None
            if s + 1 < spw:
                cp_i = pltpu.async_copy(
                    x_hbm.at[pl.ds(base + (s + 1) * _W, _W)],
                    idx_bufs[1 - p], sem_i)
            if pending[p] is not None:
                pending[p].wait()
            pltpu.async_copy(table_hbm.at[ib], yb, sem_g).wait()
            off = base + s * _W
            pending[p] = pltpu.async_copy(
                yb, y_hbm.at[pl.ds(off, _W), :], out_sems[p])
            if cp_i is not None:
                cp_i.wait()
        for pend in pending:
            if pend is not None:
                pend.wait()

    return row_kernel(x_half, table_rm)


def kernel(x, table, b, c):
    n, k = x.shape
    num = n * k
    v, dim = table.shape
    half = num // 2
    kh = k // 2

    x_flat = x.T.reshape(num)

    bsc, csc = _gather_bc(x_flat, b, c, num)

    table_rm = _transpose_table(table.T)

    del half, kh
    ysc = _gather_rows(x_flat, table_rm, num)
    y_p = _transpose_y(ysc.reshape(k, n, 128), dim, 0, k)

    y = jnp.transpose(y_p, (2, 0, 1))
    b_out = bsc.reshape(k, n).T
    c_out = csc.reshape(k, n).T
    return (y, b_out, c_out)


# speedup vs baseline: 1.5123x; 1.1795x over previous
import jax
import jax.numpy as jnp
from jax import lax
from jax.experimental import pallas as pl
from jax.experimental.pallas import tpu as pltpu
from jax.experimental.pallas import tpu_sc as plsc

_W = 256
_WB = 1024
_NW = 32
_TBLK = 8192
_YBLK = 8192

_MESH = plsc.VectorSubcoreMesh(core_axis_name="core",
                               subcore_axis_name="subcore")
_SC_PARAMS = pltpu.CompilerParams(use_tc_tiling_on_sc=False)


def _transpose_table(table_t):
    dim, v = table_t.shape
    grid = (v + _TBLK - 1) // _TBLK

    def body(in_ref, out_ref):
        out_ref[:, :dim] = in_ref[...].T

    return pl.pallas_call(
        body,
        grid=(grid,),
        in_specs=[pl.BlockSpec((dim, _TBLK), lambda g: (0, g))],
        out_specs=pl.BlockSpec((_TBLK, 128), lambda g: (g, 0)),
        out_shape=jax.ShapeDtypeStruct((v, 128), table_t.dtype),
        compiler_params=pltpu.CompilerParams(
            dimension_semantics=("parallel",)),
    )(table_t)


def _transpose_y(y3, dim, j0, nj):
    n = y3.shape[1]

    def body(in_ref, out_ref):
        out_ref[0] = in_ref[0][:, :dim].T

    return pl.pallas_call(
        body,
        grid=(nj, n // _YBLK),
        in_specs=[pl.BlockSpec((1, _YBLK, 128),
                               lambda j, i: (j + j0, i, 0))],
        out_specs=pl.BlockSpec((1, dim, _YBLK), lambda j, i: (j, 0, i)),
        out_shape=jax.ShapeDtypeStruct((nj, dim, n), y3.dtype),
        compiler_params=pltpu.CompilerParams(
            dimension_semantics=("parallel", "parallel")),
    )(y3)


def _gather_bc(x_flat, b, c, num):
    spw = num // (_WB * _NW)

    @pl.kernel(
        out_type=(
            jax.ShapeDtypeStruct((num,), b.dtype),
            jax.ShapeDtypeStruct((num,), c.dtype),
        ),
        mesh=_MESH,
        scratch_types=[
            pltpu.VMEM((_WB,), jnp.int32), pltpu.VMEM((_WB,), jnp.int32),
            pltpu.VMEM((_WB,), jnp.float32), pltpu.VMEM((_WB,), jnp.float32),
            pltpu.VMEM((_WB,), jnp.float32), pltpu.VMEM((_WB,), jnp.float32),
            pltpu.SemaphoreType.DMA, pltpu.SemaphoreType.DMA,
            pltpu.SemaphoreType.DMA, pltpu.SemaphoreType.DMA,
        ],
        compiler_params=_SC_PARAMS,
    )
    def bc_kernel(x_hbm, b_hbm, c_hbm, bo_hbm, co_hbm,
                  idx_a, idx_b, b_a, b_b, c_a, c_b,
                  sem_g, sem_oa, sem_ob, sem_i):
        wid = lax.axis_index("subcore") * 2 + lax.axis_index("core")
        base = wid * spw * _WB
        idx_bufs = (idx_a, idx_b)
        b_bufs = (b_a, b_b)
        c_bufs = (c_a, c_b)
        out_sems = (sem_oa, sem_ob)
        pending = [None, None]

        pltpu.async_copy(x_hbm.at[pl.ds(base, _WB)], idx_a, sem_i).wait()
        for s in range(spw):
            p = s % 2
            ib, bb, cb = idx_bufs[p], b_bufs[p], c_bufs[p]
            cp_i = None
            if s + 1 < spw:
                cp_i = pltpu.async_copy(
                    x_hbm.at[pl.ds(base + (s + 1) * _WB, _WB)],
                    idx_bufs[1 - p], sem_i)
            if pending[p] is not None:
                for h in pending[p]:
                    h.wait()
            g_b = pltpu.async_copy(b_hbm.at[ib], bb, sem_g)
            g_c = pltpu.async_copy(c_hbm.at[ib], cb, sem_g)
            g_b.wait()
            g_c.wait()
            off = base + s * _WB
            pending[p] = (
                pltpu.async_copy(bb, bo_hbm.at[pl.ds(off, _WB)], out_sems[p]),
                pltpu.async_copy(cb, co_hbm.at[pl.ds(off, _WB)], out_sems[p]),
            )
            if cp_i is not None:
                cp_i.wait()
        for pend in pending:
            if pend is not None:
                for h in pend:
                    h.wait()

    return bc_kernel(x_flat, b, c)


def _gather_rows(x_half, table_rm, half):
    spw = half // (_W * _NW)

    @pl.kernel(
        out_type=jax.ShapeDtypeStruct((half, 128), table_rm.dtype),
        mesh=_MESH,
        scratch_types=[
            pltpu.VMEM((_W,), jnp.int32), pltpu.VMEM((_W,), jnp.int32),
            pltpu.VMEM((_W, 128), jnp.float32),
            pltpu.VMEM((_W, 128), jnp.float32),
            pltpu.SemaphoreType.DMA, pltpu.SemaphoreType.DMA,
            pltpu.SemaphoreType.DMA, pltpu.SemaphoreType.DMA,
        ],
        compiler_params=_SC_PARAMS,
    )
    def row_kernel(x_hbm, table_hbm, y_hbm,
                   idx_a, idx_b, y_a, y_b, sem_g, sem_oa, sem_ob, sem_i):
        wid = lax.axis_index("subcore") * 2 + lax.axis_index("core")
        base = wid * spw * _W
        idx_bufs = (idx_a, idx_b)
        y_bufs = (y_a, y_b)
        out_sems = (sem_oa, sem_ob)
        pending = [None, None]

        pltpu.async_copy(x_hbm.at[pl.ds(base, _W)], idx_a, sem_i).wait()
        for s in range(spw):
            p = s % 2
            ib, yb = idx_bufs[p], y_bufs[p]
            cp_i = None
            if s + 1 < spw:
                cp_i = pltpu.async_copy(
                    x_hbm.at[pl.ds(base + (s + 1) * _W, _W)],
                    idx_bufs[1 - p], sem_i)
            if pending[p] is not None:
                pending[p].wait()
            pltpu.async_copy(table_hbm.at[ib], yb, sem_g).wait()
            off = base + s * _W
            pending[p] = pltpu.async_copy(
                yb, y_hbm.at[pl.ds(off, _W), :], out_sems[p])
            if cp_i is not None:
                cp_i.wait()
        for pend in pending:
            if pend is not None:
                pend.wait()

    return row_kernel(x_half, table_rm)


def kernel(x, table, b, c):
    n, k = x.shape
    num = n * k
    v, dim = table.shape
    half = num // 2
    kh = k // 2

    x_flat = x.T.reshape(num)

    bsc, csc = _gather_bc(x_flat, b, c, num)

    table_rm = _transpose_table(table.T)

    del half, kh
    ysc = _gather_rows(x_flat, table_rm, num)
    y_p = _transpose_y(ysc.reshape(k, n, 128), dim, 0, k)

    y = jnp.transpose(y_p, (2, 0, 1))
    b_out = bsc.reshape(k, n).T
    c_out = csc.reshape(k, n).T
    return (y, b_out, c_out)


# speedup vs baseline: 1.6372x; 1.0826x over previous
import jax
import jax.numpy as jnp
from jax import lax
from jax.experimental import pallas as pl
from jax.experimental.pallas import tpu as pltpu
from jax.experimental.pallas import tpu_sc as plsc

_W = 256
_WB = 1024
_NW = 32
_TBLK = 16384
_YBLK = 16384

_MESH = plsc.VectorSubcoreMesh(core_axis_name="core",
                               subcore_axis_name="subcore")
_SC_PARAMS = pltpu.CompilerParams(use_tc_tiling_on_sc=False)


def _transpose_table(table_t):
    dim, v = table_t.shape
    grid = (v + _TBLK - 1) // _TBLK

    def body(in_ref, out_ref):
        out_ref[:, :dim] = in_ref[...].T

    return pl.pallas_call(
        body,
        grid=(grid,),
        in_specs=[pl.BlockSpec((dim, _TBLK), lambda g: (0, g))],
        out_specs=pl.BlockSpec((_TBLK, 128), lambda g: (g, 0)),
        out_shape=jax.ShapeDtypeStruct((v, 128), table_t.dtype),
        compiler_params=pltpu.CompilerParams(
            dimension_semantics=("parallel",)),
    )(table_t)


def _transpose_y(y3, dim, j0, nj):
    n = y3.shape[1]

    def body(in_ref, out_ref):
        out_ref[0] = in_ref[0][:, :dim].T

    return pl.pallas_call(
        body,
        grid=(nj, n // _YBLK),
        in_specs=[pl.BlockSpec((1, _YBLK, 128),
                               lambda j, i: (j + j0, i, 0))],
        out_specs=pl.BlockSpec((1, dim, _YBLK), lambda j, i: (j, 0, i)),
        out_shape=jax.ShapeDtypeStruct((nj, dim, n), y3.dtype),
        compiler_params=pltpu.CompilerParams(
            dimension_semantics=("parallel", "parallel")),
    )(y3)


def _gather_bc(x_flat, b, c, num):
    spw = num // (_WB * _NW)

    @pl.kernel(
        out_type=(
            jax.ShapeDtypeStruct((num,), b.dtype),
            jax.ShapeDtypeStruct((num,), c.dtype),
        ),
        mesh=_MESH,
        scratch_types=[
            pltpu.VMEM((_WB,), jnp.int32), pltpu.VMEM((_WB,), jnp.int32),
            pltpu.VMEM((_WB,), jnp.float32), pltpu.VMEM((_WB,), jnp.float32),
            pltpu.VMEM((_WB,), jnp.float32), pltpu.VMEM((_WB,), jnp.float32),
            pltpu.SemaphoreType.DMA, pltpu.SemaphoreType.DMA,
            pltpu.SemaphoreType.DMA, pltpu.SemaphoreType.DMA,
        ],
        compiler_params=_SC_PARAMS,
    )
    def bc_kernel(x_hbm, b_hbm, c_hbm, bo_hbm, co_hbm,
                  idx_a, idx_b, b_a, b_b, c_a, c_b,
                  sem_g, sem_oa, sem_ob, sem_i):
        wid = lax.axis_index("subcore") * 2 + lax.axis_index("core")
        base = wid * spw * _WB
        idx_bufs = (idx_a, idx_b)
        b_bufs = (b_a, b_b)
        c_bufs = (c_a, c_b)
        out_sems = (sem_oa, sem_ob)
        pending = [None, None]

        pltpu.async_copy(x_hbm.at[pl.ds(base, _WB)], idx_a, sem_i).wait()
        for s in range(spw):
            p = s % 2
            ib, bb, cb = idx_bufs[p], b_bufs[p], c_bufs[p]
            cp_i = None
            if s + 1 < spw:
                cp_i = pltpu.async_copy(
                    x_hbm.at[pl.ds(base + (s + 1) * _WB, _WB)],
                    idx_bufs[1 - p], sem_i)
            if pending[p] is not None:
                for h in pending[p]:
                    h.wait()
            g_b = pltpu.async_copy(b_hbm.at[ib], bb, sem_g)
            g_c = pltpu.async_copy(c_hbm.at[ib], cb, sem_g)
            g_b.wait()
            g_c.wait()
            off = base + s * _WB
            pending[p] = (
                pltpu.async_copy(bb, bo_hbm.at[pl.ds(off, _WB)], out_sems[p]),
                pltpu.async_copy(cb, co_hbm.at[pl.ds(off, _WB)], out_sems[p]),
            )
            if cp_i is not None:
                cp_i.wait()
        for pend in pending:
            if pend is not None:
                for h in pend:
                    h.wait()

    return bc_kernel(x_flat, b, c)


def _gather_rows(x_half, table_rm, half):
    spw = half // (_W * _NW)

    @pl.kernel(
        out_type=jax.ShapeDtypeStruct((half, 128), table_rm.dtype),
        mesh=_MESH,
        scratch_types=[
            pltpu.VMEM((_W,), jnp.int32), pltpu.VMEM((_W,), jnp.int32),
            pltpu.VMEM((_W, 128), jnp.float32),
            pltpu.VMEM((_W, 128), jnp.float32),
            pltpu.SemaphoreType.DMA, pltpu.SemaphoreType.DMA,
            pltpu.SemaphoreType.DMA, pltpu.SemaphoreType.DMA,
        ],
        compiler_params=_SC_PARAMS,
    )
    def row_kernel(x_hbm, table_hbm, y_hbm,
                   idx_a, idx_b, y_a, y_b, sem_g, sem_oa, sem_ob, sem_i):
        wid = lax.axis_index("subcore") * 2 + lax.axis_index("core")
        base = wid * spw * _W
        idx_bufs = (idx_a, idx_b)
        y_bufs = (y_a, y_b)
        out_sems = (sem_oa, sem_ob)
        pending = [None, None]

        pltpu.async_copy(x_hbm.at[pl.ds(base, _W)], idx_a, sem_i).wait()
        for s in range(spw):
            p = s % 2
            ib, yb = idx_bufs[p], y_bufs[p]
            cp_i = None
            if s + 1 < spw:
                cp_i = pltpu.async_copy(
                    x_hbm.at[pl.ds(base + (s + 1) * _W, _W)],
                    idx_bufs[1 - p], sem_i)
            if pending[p] is not None:
                pending[p].wait()
            pltpu.async_copy(table_hbm.at[ib], yb, sem_g).wait()
            off = base + s * _W
            pending[p] = pltpu.async_copy(
                yb, y_hbm.at[pl.ds(off, _W), :], out_sems[p])
            if cp_i is not None:
                cp_i.wait()
        for pend in pending:
            if pend is not None:
                pend.wait()

    return row_kernel(x_half, table_rm)


def kernel(x, table, b, c):
    n, k = x.shape
    num = n * k
    v, dim = table.shape
    half = num // 2
    kh = k // 2

    x_flat = x.T.reshape(num)

    bsc, csc = _gather_bc(x_flat, b, c, num)

    table_rm = _transpose_table(table.T)

    del half, kh
    ysc = _gather_rows(x_flat, table_rm, num)
    y_p = _transpose_y(ysc.reshape(k, n, 128), dim, 0, k)

    y = jnp.transpose(y_p, (2, 0, 1))
    b_out = bsc.reshape(k, n).T
    c_out = csc.reshape(k, n).T
    return (y, b_out, c_out)


# speedup vs baseline: 1.6536x; 1.0100x over previous
import jax
import jax.numpy as jnp
from jax import lax
from jax.experimental import pallas as pl
from jax.experimental.pallas import tpu as pltpu
from jax.experimental.pallas import tpu_sc as plsc

_W = 256
_WB = 1024
_NW = 32
_TBLK = 32768
_YBLK = 16384

_MESH = plsc.VectorSubcoreMesh(core_axis_name="core",
                               subcore_axis_name="subcore")
_SC_PARAMS = pltpu.CompilerParams(use_tc_tiling_on_sc=False)


def _transpose_table(table_t):
    dim, v = table_t.shape
    grid = (v + _TBLK - 1) // _TBLK

    def body(in_ref, out_ref):
        out_ref[:, :dim] = in_ref[...].T

    return pl.pallas_call(
        body,
        grid=(grid,),
        in_specs=[pl.BlockSpec((dim, _TBLK), lambda g: (0, g))],
        out_specs=pl.BlockSpec((_TBLK, 128), lambda g: (g, 0)),
        out_shape=jax.ShapeDtypeStruct((v, 128), table_t.dtype),
        compiler_params=pltpu.CompilerParams(
            dimension_semantics=("parallel",)),
    )(table_t)


def _transpose_y(y3, dim, j0, nj):
    n = y3.shape[1]

    def body(in_ref, out_ref):
        out_ref[0] = in_ref[0][:, :dim].T

    return pl.pallas_call(
        body,
        grid=(nj, n // _YBLK),
        in_specs=[pl.BlockSpec((1, _YBLK, 128),
                               lambda j, i: (j + j0, i, 0))],
        out_specs=pl.BlockSpec((1, dim, _YBLK), lambda j, i: (j, 0, i)),
        out_shape=jax.ShapeDtypeStruct((nj, dim, n), y3.dtype),
        compiler_params=pltpu.CompilerParams(
            dimension_semantics=("parallel", "parallel")),
    )(y3)


def _gather_bc(x_flat, b, c, num):
    spw = num // (_WB * _NW)

    @pl.kernel(
        out_type=(
            jax.ShapeDtypeStruct((num,), b.dtype),
            jax.ShapeDtypeStruct((num,), c.dtype),
        ),
        mesh=_MESH,
        scratch_types=[
            pltpu.VMEM((_WB,), jnp.int32), pltpu.VMEM((_WB,), jnp.int32),
            pltpu.VMEM((_WB,), jnp.float32), pltpu.VMEM((_WB,), jnp.float32),
            pltpu.VMEM((_WB,), jnp.float32), pltpu.VMEM((_WB,), jnp.float32),
            pltpu.SemaphoreType.DMA, pltpu.SemaphoreType.DMA,
            pltpu.SemaphoreType.DMA, pltpu.SemaphoreType.DMA,
        ],
        compiler_params=_SC_PARAMS,
    )
    def bc_kernel(x_hbm, b_hbm, c_hbm, bo_hbm, co_hbm,
                  idx_a, idx_b, b_a, b_b, c_a, c_b,
                  sem_g, sem_oa, sem_ob, sem_i):
        wid = lax.axis_index("subcore") * 2 + lax.axis_index("core")
        base = wid * spw * _WB
        idx_bufs = (idx_a, idx_b)
        b_bufs = (b_a, b_b)
        c_bufs = (c_a, c_b)
        out_sems = (sem_oa, sem_ob)
        pending = [None, None]

        pltpu.async_copy(x_hbm.at[pl.ds(base, _WB)], idx_a, sem_i).wait()
        for s in range(spw):
            p = s % 2
            ib, bb, cb = idx_bufs[p], b_bufs[p], c_bufs[p]
            cp_i = None
            if s + 1 < spw:
                cp_i = pltpu.async_copy(
                    x_hbm.at[pl.ds(base + (s + 1) * _WB, _WB)],
                    idx_bufs[1 - p], sem_i)
            if pending[p] is not None:
                for h in pending[p]:
                    h.wait()
            g_b = pltpu.async_copy(b_hbm.at[ib], bb, sem_g)
            g_c = pltpu.async_copy(c_hbm.at[ib], cb, sem_g)
            g_b.wait()
            g_c.wait()
            off = base + s * _WB
            pending[p] = (
                pltpu.async_copy(bb, bo_hbm.at[pl.ds(off, _WB)], out_sems[p]),
                pltpu.async_copy(cb, co_hbm.at[pl.ds(off, _WB)], out_sems[p]),
            )
            if cp_i is not None:
                cp_i.wait()
        for pend in pending:
            if pend is not None:
                for h in pend:
                    h.wait()

    return bc_kernel(x_flat, b, c)


def _gather_rows(x_half, table_rm, half):
    spw = half // (_W * _NW)

    @pl.kernel(
        out_type=jax.ShapeDtypeStruct((half, 128), table_rm.dtype),
        mesh=_MESH,
        scratch_types=[
            pltpu.VMEM((_W,), jnp.int32), pltpu.VMEM((_W,), jnp.int32),
            pltpu.VMEM((_W, 128), jnp.float32),
            pltpu.VMEM((_W, 128), jnp.float32),
            pltpu.SemaphoreType.DMA, pltpu.SemaphoreType.DMA,
            pltpu.SemaphoreType.DMA, pltpu.SemaphoreType.DMA,
        ],
        compiler_params=_SC_PARAMS,
    )
    def row_kernel(x_hbm, table_hbm, y_hbm,
                   idx_a, idx_b, y_a, y_b, sem_g, sem_oa, sem_ob, sem_i):
        wid = lax.axis_index("subcore") * 2 + lax.axis_index("core")
        base = wid * spw * _W
        idx_bufs = (idx_a, idx_b)
        y_bufs = (y_a, y_b)
        out_sems = (sem_oa, sem_ob)
        pending = [None, None]

        pltpu.async_copy(x_hbm.at[pl.ds(base, _W)], idx_a, sem_i).wait()
        for s in range(spw):
            p = s % 2
            ib, yb = idx_bufs[p], y_bufs[p]
            cp_i = None
            if s + 1 < spw:
                cp_i = pltpu.async_copy(
                    x_hbm.at[pl.ds(base + (s + 1) * _W, _W)],
                    idx_bufs[1 - p], sem_i)
            if pending[p] is not None:
                pending[p].wait()
            pltpu.async_copy(table_hbm.at[ib], yb, sem_g).wait()
            off = base + s * _W
            pending[p] = pltpu.async_copy(
                yb, y_hbm.at[pl.ds(off, _W), :], out_sems[p])
            if cp_i is not None:
                cp_i.wait()
        for pend in pending:
            if pend is not None:
                pend.wait()

    return row_kernel(x_half, table_rm)


def kernel(x, table, b, c):
    n, k = x.shape
    num = n * k
    v, dim = table.shape
    half = num // 2
    kh = k // 2

    x_flat = x.T.reshape(num)

    bsc, csc = _gather_bc(x_flat, b, c, num)

    table_rm = _transpose_table(table.T)

    del half, kh
    ysc = _gather_rows(x_flat, table_rm, num)
    y_p = _transpose_y(ysc.reshape(k, n, 128), dim, 0, k)

    y = jnp.transpose(y_p, (2, 0, 1))
    b_out = bsc.reshape(k, n).T
    c_out = csc.reshape(k, n).T
    return (y, b_out, c_out)
